# baseline (speedup 1.0000x reference)
import functools

import jax
import jax.numpy as jnp
from jax import lax
from jax.experimental import pallas as pl
from jax.experimental.pallas import tpu as pltpu
from jax.experimental.pallas import tpu_sc as plsc

NC = 2
NS = 16
L = 16
NW = NC * NS

CH = 128

_SC_MESH = dict(core_axis_name="c", subcore_axis_name="s")


def _mesh():
  return plsc.VectorSubcoreMesh(**_SC_MESH)


def _sc_degree(dst, ew, n_nodes):
  e = dst.shape[0]
  ew_per = e // NW
  rpt = n_nodes // NS

  @functools.partial(
      pl.kernel,
      out_type=jax.ShapeDtypeStruct((NC, n_nodes), jnp.float32),
      mesh=_mesh(),
      scratch_types=(
          pltpu.VMEM((CH,), jnp.int32),
          pltpu.VMEM((CH,), jnp.float32),
          pltpu.VMEM((rpt,), jnp.float32),
          pltpu.VMEM_SHARED((n_nodes,), jnp.float32),
      ),
  )
  def k(dst_hbm, ew_hbm, out_hbm, idx, val, zbuf, deg_sh):
    c = lax.axis_index("c")
    s = lax.axis_index("s")
    w = c * NS + s

    @pl.loop(0, rpt // L)
    def _(i):
      zbuf[pl.ds(i * L, L)] = jnp.zeros((L,), jnp.float32)

    pltpu.sync_copy(zbuf, deg_sh.at[pl.ds(s * rpt, rpt)])
    plsc.subcore_barrier()

    base = w * ew_per

    @pl.loop(0, ew_per // CH)
    def _(g):
      o = base + g * CH
      pltpu.sync_copy(dst_hbm.at[pl.ds(o, CH)], idx)
      pltpu.sync_copy(ew_hbm.at[pl.ds(o, CH)], val)
      pltpu.sync_copy(val, deg_sh.at[idx], add=True)

    plsc.subcore_barrier()
    pltpu.sync_copy(deg_sh.at[pl.ds(s * rpt, rpt)],
                    out_hbm.at[c, pl.ds(s * rpt, rpt)])

  return k(dst, ew)


def _sc_msgpass(hp, src, dst, ew):
  n_nodes, d = hp.shape
  e = src.shape[0]
  ew_per = e // NW
  rpt = n_nodes // NS

  @functools.partial(
      pl.kernel,
      out_type=jax.ShapeDtypeStruct((NC, n_nodes, d), jnp.float32),
      mesh=_mesh(),
      scratch_types=(
          pltpu.VMEM((CH,), jnp.int32),
          pltpu.VMEM((CH,), jnp.int32),
          pltpu.VMEM((CH,), jnp.float32),
          pltpu.VMEM((CH, d), jnp.float32),
          pltpu.VMEM_SHARED((n_nodes, d), jnp.float32),
          pltpu.SemaphoreType.DMA,
      ),
      compiler_params=pltpu.CompilerParams(use_tc_tiling_on_sc=False,
                                           needs_layout_passes=False),
  )
  def k(hp_hbm, src_hbm, dst_hbm, ew_hbm, out_hbm,
        sidx, didx, wv, rows, acc_sh, sem):
    c = lax.axis_index("c")
    s = lax.axis_index("s")
    w = c * NS + s

    @pl.loop(0, CH)
    def _(i):
      for j in range(d // L):
        rows[i, pl.ds(j * L, L)] = jnp.zeros((L,), jnp.float32)

    @pl.loop(0, rpt // CH)
    def _(t):
      pltpu.sync_copy(rows, acc_sh.at[pl.ds(s * rpt + t * CH, CH), :])

    plsc.subcore_barrier()

    base = w * ew_per

    @pl.loop(0, ew_per // CH)
    def _(g):
      o = base + g * CH
      pltpu.sync_copy(src_hbm.at[pl.ds(o, CH)], sidx)
      pltpu.sync_copy(dst_hbm.at[pl.ds(o, CH)], didx)
      pltpu.sync_copy(ew_hbm.at[pl.ds(o, CH)], wv)
      pltpu.async_copy(hp_hbm.at[sidx], rows, sem).wait()

      lanes = lax.iota(jnp.int32, L)

      @pl.loop(0, CH // L)
      def _(i):
        w16 = wv[pl.ds(i * L, L)]
        ridx = lanes + i * L

        @pl.loop(0, d)
        def _(j):
          cidx = jnp.full((L,), j, jnp.int32)
          col = plsc.load_gather(rows, [ridx, cidx])
          plsc.store_scatter(rows, [ridx, cidx], col * w16)

      pltpu.sync_copy(rows, acc_sh.at[didx], add=True)

    plsc.subcore_barrier()
    pltpu.sync_copy(acc_sh.at[pl.ds(s * rpt, rpt), :],
                    out_hbm.at[c, pl.ds(s * rpt, rpt), :])

  return k(hp, src, dst, ew)


def _sc_adj(src, dst, ew, n_nodes):
  e = src.shape[0]
  nn = n_nodes * n_nodes
  e_per_tile = e // NS
  zregion = nn // NW
  half = n_nodes // NC

  @functools.partial(
      pl.kernel,
      out_type=jax.ShapeDtypeStruct((nn + 64,), jnp.float32),
      mesh=_mesh(),
      scratch_types=(
          pltpu.VMEM((CH,), jnp.int32),
          pltpu.VMEM((CH,), jnp.int32),
          pltpu.VMEM((CH,), jnp.int32),
          pltpu.VMEM((CH,), jnp.float32),
          pltpu.VMEM((CH * L,), jnp.float32),
          pltpu.SemaphoreType.DMA,
      ),
  )
  def k(src_hbm, dst_hbm, ew_hbm, out_hbm, sidx, didx, fidx, wv, zbuf, sem):
    c = lax.axis_index("c")
    s = lax.axis_index("s")
    w = c * NS + s

    @pl.loop(0, CH * L // L)
    def _(i):
      zbuf[pl.ds(i * L, L)] = jnp.zeros((L,), jnp.float32)

    @pl.loop(0, zregion // (CH * L))
    def _(t):
      pltpu.sync_copy(zbuf, out_hbm.at[pl.ds(w * zregion + t * (CH * L),
                                             CH * L)])

    plsc.subcore_barrier()

    lo = c * half
    base = s * e_per_tile

    @pl.loop(0, e_per_tile // CH)
    def _(g):
      o = base + g * CH
      pltpu.sync_copy(src_hbm.at[pl.ds(o, CH)], sidx)
      pltpu.sync_copy(dst_hbm.at[pl.ds(o, CH)], didx)
      pltpu.sync_copy(ew_hbm.at[pl.ds(o, CH)], wv)

      @pl.loop(0, CH // L)
      def _(j):
        sv = sidx[pl.ds(j * L, L)]
        dv = didx[pl.ds(j * L, L)]
        flat = sv * n_nodes + dv
        mask = (sv >= lo) & (sv < lo + half)
        fidx[pl.ds(j * L, L)] = jnp.where(mask, flat, nn)

      pltpu.async_copy(wv, out_hbm.at[fidx], sem).wait()

  return k(src, dst, ew)


def _tc_layer1(x, w1, b1, degp_t):
  n_nodes, d = x.shape
  r = 256
  grid = n_nodes // r

  def body(x_ref, w_ref, b_ref, deg_ref, hp_ref, dinv_ref):
    deg = jnp.sum(deg_ref[...], axis=1, keepdims=True) + 1.0
    dinv = jnp.where(deg > 0, lax.rsqrt(deg), 0.0)
    h = jnp.dot(x_ref[...], w_ref[...], preferred_element_type=jnp.float32)
    h = h + b_ref[...]
    hp_ref[...] = h * dinv
    dinv_ref[...] = dinv

  return pl.pallas_call(
      body,
      grid=(grid,),
      in_specs=[
          pl.BlockSpec((r, d), lambda i: (i, 0)),
          pl.BlockSpec((d, d), lambda i: (0, 0)),
          pl.BlockSpec((1, d), lambda i: (0, 0)),
          pl.BlockSpec((r, NC), lambda i: (i, 0)),
      ],
      out_specs=[
          pl.BlockSpec((r, d), lambda i: (i, 0)),
          pl.BlockSpec((r, 1), lambda i: (i, 0)),
      ],
      out_shape=[
          jax.ShapeDtypeStruct((n_nodes, d), jnp.float32),
          jax.ShapeDtypeStruct((n_nodes, 1), jnp.float32),
      ],
  )(x, w1, b1, degp_t)


def _tc_layer2(acc1, hp1, dinv, w2, b2):
  _, n_nodes, d = acc1.shape
  r = 256
  grid = n_nodes // r

  def body(acc_ref, hp_ref, dinv_ref, w_ref, b_ref, out_ref):
    dinv_blk = dinv_ref[...]
    h1 = dinv_blk * (acc_ref[0] + acc_ref[1] + hp_ref[...])
    x2 = jnp.maximum(h1, 0.0)
    h2 = jnp.dot(x2, w_ref[...], preferred_element_type=jnp.float32)
    h2 = h2 + b_ref[...]
    out_ref[...] = h2 * dinv_blk

  return pl.pallas_call(
      body,
      grid=(grid,),
      in_specs=[
          pl.BlockSpec((NC, r, d), lambda i: (0, i, 0)),
          pl.BlockSpec((r, d), lambda i: (i, 0)),
          pl.BlockSpec((r, 1), lambda i: (i, 0)),
          pl.BlockSpec((d, d), lambda i: (0, 0)),
          pl.BlockSpec((1, d), lambda i: (0, 0)),
      ],
      out_specs=pl.BlockSpec((r, d), lambda i: (i, 0)),
      out_shape=jax.ShapeDtypeStruct((n_nodes, d), jnp.float32),
  )(acc1, hp1, dinv, w2, b2)


def _tc_finalize(acc2, hp2, dinv, x):
  _, n_nodes, d = acc2.shape
  r = 256
  grid = n_nodes // r

  def body(acc_ref, hp_ref, dinv_ref, x_ref, z_ref, enc_ref):
    z = dinv_ref[...] * (acc_ref[0] + acc_ref[1] + hp_ref[...])
    z_ref[...] = z
    enc_ref[...] = z + x_ref[...]

  return pl.pallas_call(
      body,
      grid=(grid,),
      in_specs=[
          pl.BlockSpec((NC, r, d), lambda i: (0, i, 0)),
          pl.BlockSpec((r, d), lambda i: (i, 0)),
          pl.BlockSpec((r, 1), lambda i: (i, 0)),
          pl.BlockSpec((r, d), lambda i: (i, 0)),
      ],
      out_specs=[
          pl.BlockSpec((r, d), lambda i: (i, 0)),
          pl.BlockSpec((r, d), lambda i: (i, 0)),
      ],
      out_shape=[
          jax.ShapeDtypeStruct((n_nodes, d), jnp.float32),
          jax.ShapeDtypeStruct((n_nodes, d), jnp.float32),
      ],
  )(acc2, hp2, dinv, x)


def _tc_decoder(z, adj_flat):
  n_nodes, d = z.shape
  rb = 128
  grid = n_nodes // rb

  def body(z_ref, adj_ref, out_ref):
    i = pl.program_id(0)
    zr = z_ref[pl.ds(i * rb, rb), :]
    s = lax.dot_general(zr, z_ref[...],
                        dimension_numbers=(((1,), (1,)), ((), ())),
                        preferred_element_type=jnp.float32)
    adj = adj_ref[...].reshape(rb, n_nodes)
    out_ref[...] = jax.nn.sigmoid(s + adj)

  return pl.pallas_call(
      body,
      grid=(grid,),
      in_specs=[
          pl.BlockSpec((n_nodes, d), lambda i: (0, 0)),
          pl.BlockSpec((rb * n_nodes,), lambda i: (i,)),
      ],
      out_specs=pl.BlockSpec((rb, n_nodes), lambda i: (i, 0)),
      out_shape=jax.ShapeDtypeStruct((n_nodes, n_nodes), jnp.float32),
  )(z, adj_flat)


def kernel(node_features, edge_list, edge_attr, batch, W1, b1, W2, b2):
  n_nodes, d = node_features.shape
  src = edge_list[0]
  dst = edge_list[1]
  ew = edge_attr[:, 0]

  degp = _sc_degree(dst, ew, n_nodes)
  adj_flat = _sc_adj(src, dst, ew, n_nodes)

  hp1, dinv = _tc_layer1(node_features, W1, b1.reshape(1, d),
                         degp.T)
  acc1 = _sc_msgpass(hp1, src, dst, ew)
  hp2 = _tc_layer2(acc1, hp1, dinv, W2, b2.reshape(1, d))
  acc2 = _sc_msgpass(hp2, src, dst, ew)
  z, encoded = _tc_finalize(acc2, hp2, dinv, node_features)
  probs = _tc_decoder(z, adj_flat)
  return (encoded, edge_list, probs)


# speedup vs baseline: 6.8890x; 6.8890x over previous
import functools

import jax
import jax.numpy as jnp
from jax import lax
from jax.experimental import pallas as pl
from jax.experimental.pallas import tpu as pltpu
from jax.experimental.pallas import tpu_sc as plsc

NC = 2
NS = 16
L = 16
NW = NC * NS

CH = 128

_SC_MESH = dict(core_axis_name="c", subcore_axis_name="s")


def _mesh():
  return plsc.VectorSubcoreMesh(**_SC_MESH)


def _sc_degree(dst, ew, n_nodes):
  e = dst.shape[0]
  ew_per = e // NW
  rpt = n_nodes // NS

  @functools.partial(
      pl.kernel,
      out_type=jax.ShapeDtypeStruct((NC, n_nodes), jnp.float32),
      mesh=_mesh(),
      scratch_types=(
          pltpu.VMEM((CH,), jnp.int32),
          pltpu.VMEM((CH,), jnp.float32),
          pltpu.VMEM((rpt,), jnp.float32),
          pltpu.VMEM_SHARED((n_nodes,), jnp.float32),
      ),
  )
  def k(dst_hbm, ew_hbm, out_hbm, idx, val, zbuf, deg_sh):
    c = lax.axis_index("c")
    s = lax.axis_index("s")
    w = c * NS + s

    @pl.loop(0, rpt // L)
    def _(i):
      zbuf[pl.ds(i * L, L)] = jnp.zeros((L,), jnp.float32)

    pltpu.sync_copy(zbuf, deg_sh.at[pl.ds(s * rpt, rpt)])
    plsc.subcore_barrier()

    base = w * ew_per

    @pl.loop(0, ew_per // CH)
    def _(g):
      o = base + g * CH
      pltpu.sync_copy(dst_hbm.at[pl.ds(o, CH)], idx)
      pltpu.sync_copy(ew_hbm.at[pl.ds(o, CH)], val)
      pltpu.sync_copy(val, deg_sh.at[idx], add=True)

    plsc.subcore_barrier()
    pltpu.sync_copy(deg_sh.at[pl.ds(s * rpt, rpt)],
                    out_hbm.at[c, pl.ds(s * rpt, rpt)])

  return k(dst, ew)


CHM = 64


def _sc_msgpass(hp, src2d, dst2d, ew):
  n_nodes, d = hp.shape
  e = ew.shape[0]
  ew_per = e // NW
  gpw = ew_per // CHM
  rpt = n_nodes // NS

  @functools.partial(
      pl.kernel,
      out_type=jax.ShapeDtypeStruct((NC, n_nodes, d), jnp.float32),
      mesh=_mesh(),
      scratch_types=(
          pltpu.VMEM((gpw, CHM), jnp.int32),
          pltpu.VMEM((gpw, CHM), jnp.int32),
          pltpu.VMEM((ew_per,), jnp.float32),
          pltpu.VMEM((CHM, d), jnp.float32),
          pltpu.VMEM((CHM, d), jnp.float32),
          pltpu.VMEM_SHARED((n_nodes, d), jnp.float32),
          pltpu.SemaphoreType.DMA,
          pltpu.SemaphoreType.DMA,
          pltpu.SemaphoreType.DMA,
          pltpu.SemaphoreType.DMA,
      ),
      compiler_params=pltpu.CompilerParams(use_tc_tiling_on_sc=False,
                                           needs_layout_passes=False),
  )
  def k(hp_hbm, src_hbm, dst_hbm, ew_hbm, out_hbm,
        sidx, didx, wv, rows_a, rows_b, acc_sh, gsa, gsb, ssa, ssb):
    c = lax.axis_index("c")
    s = lax.axis_index("s")
    w = c * NS + s

    pltpu.sync_copy(src_hbm.at[pl.ds(w * gpw, gpw), :], sidx)
    pltpu.sync_copy(dst_hbm.at[pl.ds(w * gpw, gpw), :], didx)
    pltpu.sync_copy(ew_hbm.at[pl.ds(w * ew_per, ew_per)], wv)

    @pl.loop(0, CHM)
    def _(i):
      for j in range(d // L):
        rows_a[i, pl.ds(j * L, L)] = jnp.zeros((L,), jnp.float32)

    @pl.loop(0, rpt // CHM)
    def _(t):
      pltpu.sync_copy(rows_a, acc_sh.at[pl.ds(s * rpt + t * CHM, CHM), :])

    plsc.subcore_barrier()

    lanes = lax.iota(jnp.int32, L)

    def scale(rows, g):
      @pl.loop(0, CHM // L)
      def _(i):
        w16 = wv[pl.ds(g * CHM + i * L, L)]
        ridx = lanes + i * L

        @pl.loop(0, d, unroll=16)
        def _(j):
          cidx = jnp.full((L,), j, jnp.int32)
          col = plsc.load_gather(rows, [ridx, cidx])
          plsc.store_scatter(rows, [ridx, cidx], col * w16)

    def start_gather(rows, sem, g):
      pltpu.async_copy(hp_hbm.at[sidx.at[g]], rows, sem)

    def wait_gather(rows, sem):
      pltpu.make_async_copy(hp_hbm.at[sidx.at[0]], rows, sem).wait()

    def wait_scatter(rows, sem):
      pltpu.make_async_copy(rows, acc_sh.at[didx.at[0]], sem).wait()

    start_gather(rows_a, gsa, 0)
    start_gather(rows_b, gsb, 1)

    @pl.loop(0, gpw, step=2)
    def _(g):
      wait_gather(rows_a, gsa)
      scale(rows_a, g)
      pltpu.async_copy(rows_a, acc_sh.at[didx.at[g]], ssa, add=True)
      wait_gather(rows_b, gsb)
      scale(rows_b, g + 1)
      pltpu.async_copy(rows_b, acc_sh.at[didx.at[g + 1]], ssb, add=True)
      wait_scatter(rows_a, ssa)
      start_gather(rows_a, gsa, jnp.minimum(g + 2, gpw - 1))
      wait_scatter(rows_b, ssb)
      start_gather(rows_b, gsb, jnp.minimum(g + 3, gpw - 1))

    wait_gather(rows_a, gsa)
    wait_gather(rows_b, gsb)
    plsc.subcore_barrier()
    pltpu.sync_copy(acc_sh.at[pl.ds(s * rpt, rpt), :],
                    out_hbm.at[c, pl.ds(s * rpt, rpt), :])

  return k(hp, src2d, dst2d, ew)


JUNK = 32 * 512


def _sc_adj(src2d, dst2d, ew, n_nodes):
  e = ew.shape[0]
  nn = n_nodes * n_nodes
  e_per_tile = e // NS
  gpt = e_per_tile // CH
  zregion = nn // NW
  zch = 32768
  half = n_nodes // NC

  @functools.partial(
      pl.kernel,
      out_type=jax.ShapeDtypeStruct((nn + JUNK,), jnp.float32),
      mesh=_mesh(),
      scratch_types=(
          pltpu.VMEM((gpt, CH), jnp.int32),
          pltpu.VMEM((gpt, CH), jnp.int32),
          pltpu.VMEM((gpt, CH), jnp.int32),
          pltpu.VMEM((e_per_tile,), jnp.float32),
          pltpu.VMEM((zch,), jnp.float32),
          pltpu.SemaphoreType.DMA,
      ),
  )
  def k(src_hbm, dst_hbm, ew_hbm, out_hbm, sidx, didx, fidx, wv, zbuf, sem):
    c = lax.axis_index("c")
    s = lax.axis_index("s")
    w = c * NS + s

    pltpu.sync_copy(src_hbm.at[pl.ds(s * gpt, gpt), :], sidx)
    pltpu.sync_copy(dst_hbm.at[pl.ds(s * gpt, gpt), :], didx)
    pltpu.sync_copy(ew_hbm.at[pl.ds(s * e_per_tile, e_per_tile)], wv)

    @pl.loop(0, zch // L)
    def _(i):
      zbuf[pl.ds(i * L, L)] = jnp.zeros((L,), jnp.float32)

    @pl.loop(0, zregion // zch)
    def _(t):
      pltpu.sync_copy(zbuf, out_hbm.at[pl.ds(w * zregion + t * zch, zch)])

    plsc.subcore_barrier()

    lo = c * half
    lanes = lax.iota(jnp.int32, L)
    junk_base = nn + w * 512

    @pl.loop(0, gpt)
    def _(g):
      @pl.loop(0, CH // L)
      def _(j):
        sv = sidx[g, pl.ds(j * L, L)]
        dv = didx[g, pl.ds(j * L, L)]
        flat = sv * n_nodes + dv
        mask = (sv >= lo) & (sv < lo + half)
        junk = junk_base + ((g % 4) * 128 + j * L) + lanes
        fidx[g, pl.ds(j * L, L)] = jnp.where(mask, flat, junk)

    @pl.loop(0, gpt // 16)
    def _(t):
      @pl.loop(0, 16)
      def _(u):
        g = t * 16 + u
        pltpu.async_copy(wv.at[pl.ds(g * CH, CH)], out_hbm.at[fidx.at[g]],
                         sem)

      @pl.loop(0, 16)
      def _(u):
        pltpu.make_async_copy(wv.at[pl.ds(0, CH)], out_hbm.at[fidx.at[0]],
                              sem).wait()

  return k(src2d, dst2d, ew)


def _tc_layer1(x, w1, b1, degp_t):
  n_nodes, d = x.shape
  r = 256
  grid = n_nodes // r

  def body(x_ref, w_ref, b_ref, deg_ref, hp_ref, dinv_ref):
    deg = jnp.sum(deg_ref[...], axis=1, keepdims=True) + 1.0
    dinv = jnp.where(deg > 0, lax.rsqrt(deg), 0.0)
    h = jnp.dot(x_ref[...], w_ref[...], preferred_element_type=jnp.float32)
    h = h + b_ref[...]
    hp_ref[...] = h * dinv
    dinv_ref[...] = dinv

  return pl.pallas_call(
      body,
      grid=(grid,),
      in_specs=[
          pl.BlockSpec((r, d), lambda i: (i, 0)),
          pl.BlockSpec((d, d), lambda i: (0, 0)),
          pl.BlockSpec((1, d), lambda i: (0, 0)),
          pl.BlockSpec((r, NC), lambda i: (i, 0)),
      ],
      out_specs=[
          pl.BlockSpec((r, d), lambda i: (i, 0)),
          pl.BlockSpec((r, 1), lambda i: (i, 0)),
      ],
      out_shape=[
          jax.ShapeDtypeStruct((n_nodes, d), jnp.float32),
          jax.ShapeDtypeStruct((n_nodes, 1), jnp.float32),
      ],
  )(x, w1, b1, degp_t)


def _tc_layer2(acc1, hp1, dinv, w2, b2):
  _, n_nodes, d = acc1.shape
  r = 256
  grid = n_nodes // r

  def body(acc_ref, hp_ref, dinv_ref, w_ref, b_ref, out_ref):
    dinv_blk = dinv_ref[...]
    h1 = dinv_blk * (acc_ref[0] + acc_ref[1] + hp_ref[...])
    x2 = jnp.maximum(h1, 0.0)
    h2 = jnp.dot(x2, w_ref[...], preferred_element_type=jnp.float32)
    h2 = h2 + b_ref[...]
    out_ref[...] = h2 * dinv_blk

  return pl.pallas_call(
      body,
      grid=(grid,),
      in_specs=[
          pl.BlockSpec((NC, r, d), lambda i: (0, i, 0)),
          pl.BlockSpec((r, d), lambda i: (i, 0)),
          pl.BlockSpec((r, 1), lambda i: (i, 0)),
          pl.BlockSpec((d, d), lambda i: (0, 0)),
          pl.BlockSpec((1, d), lambda i: (0, 0)),
      ],
      out_specs=pl.BlockSpec((r, d), lambda i: (i, 0)),
      out_shape=jax.ShapeDtypeStruct((n_nodes, d), jnp.float32),
  )(acc1, hp1, dinv, w2, b2)


def _tc_finalize(acc2, hp2, dinv, x):
  _, n_nodes, d = acc2.shape
  r = 256
  grid = n_nodes // r

  def body(acc_ref, hp_ref, dinv_ref, x_ref, z_ref, enc_ref):
    z = dinv_ref[...] * (acc_ref[0] + acc_ref[1] + hp_ref[...])
    z_ref[...] = z
    enc_ref[...] = z + x_ref[...]

  return pl.pallas_call(
      body,
      grid=(grid,),
      in_specs=[
          pl.BlockSpec((NC, r, d), lambda i: (0, i, 0)),
          pl.BlockSpec((r, d), lambda i: (i, 0)),
          pl.BlockSpec((r, 1), lambda i: (i, 0)),
          pl.BlockSpec((r, d), lambda i: (i, 0)),
      ],
      out_specs=[
          pl.BlockSpec((r, d), lambda i: (i, 0)),
          pl.BlockSpec((r, d), lambda i: (i, 0)),
      ],
      out_shape=[
          jax.ShapeDtypeStruct((n_nodes, d), jnp.float32),
          jax.ShapeDtypeStruct((n_nodes, d), jnp.float32),
      ],
  )(acc2, hp2, dinv, x)


def _tc_decoder(z, adj_flat):
  n_nodes, d = z.shape
  rb = 128
  grid = n_nodes // rb

  def body(z_ref, adj_ref, out_ref):
    i = pl.program_id(0)
    zr = z_ref[pl.ds(i * rb, rb), :]
    s = lax.dot_general(zr, z_ref[...],
                        dimension_numbers=(((1,), (1,)), ((), ())),
                        preferred_element_type=jnp.float32)
    adj = adj_ref[...].reshape(rb, n_nodes)
    out_ref[...] = jax.nn.sigmoid(s + adj)

  return pl.pallas_call(
      body,
      grid=(grid,),
      in_specs=[
          pl.BlockSpec((n_nodes, d), lambda i: (0, 0)),
          pl.BlockSpec((rb * n_nodes,), lambda i: (i,)),
      ],
      out_specs=pl.BlockSpec((rb, n_nodes), lambda i: (i, 0)),
      out_shape=jax.ShapeDtypeStruct((n_nodes, n_nodes), jnp.float32),
  )(z, adj_flat)


def kernel(node_features, edge_list, edge_attr, batch, W1, b1, W2, b2):
  n_nodes, d = node_features.shape
  src = edge_list[0]
  dst = edge_list[1]
  ew = edge_attr[:, 0]
  src2d = src.reshape(-1, CH)
  dst2d = dst.reshape(-1, CH)
  src2m = src.reshape(-1, CHM)
  dst2m = dst.reshape(-1, CHM)

  degp = _sc_degree(dst, ew, n_nodes)
  adj_flat = _sc_adj(src2d, dst2d, ew, n_nodes)

  hp1, dinv = _tc_layer1(node_features, W1, b1.reshape(1, d),
                         degp.T)
  acc1 = _sc_msgpass(hp1, src2m, dst2m, ew)
  hp2 = _tc_layer2(acc1, hp1, dinv, W2, b2.reshape(1, d))
  acc2 = _sc_msgpass(hp2, src2m, dst2m, ew)
  z, encoded = _tc_finalize(acc2, hp2, dinv, node_features)
  probs = _tc_decoder(z, adj_flat)
  return (encoded, edge_list, probs)


# speedup vs baseline: 40.3868x; 5.8625x over previous
import functools

import jax
import jax.numpy as jnp
from jax import lax
from jax.experimental import pallas as pl
from jax.experimental.pallas import tpu as pltpu
from jax.experimental.pallas import tpu_sc as plsc

NC = 2
NS = 16
L = 16
NW = NC * NS

CH = 128

_SC_MESH = dict(core_axis_name="c", subcore_axis_name="s")


def _mesh():
  return plsc.VectorSubcoreMesh(**_SC_MESH)


def _sc_degree(dst, ew, n_nodes):
  e = dst.shape[0]
  ew_per = e // NW
  rpt = n_nodes // NS

  @functools.partial(
      pl.kernel,
      out_type=jax.ShapeDtypeStruct((NC, n_nodes), jnp.float32),
      mesh=_mesh(),
      scratch_types=(
          pltpu.VMEM((CH,), jnp.int32),
          pltpu.VMEM((CH,), jnp.float32),
          pltpu.VMEM((rpt,), jnp.float32),
          pltpu.VMEM_SHARED((n_nodes,), jnp.float32),
      ),
  )
  def k(dst_hbm, ew_hbm, out_hbm, idx, val, zbuf, deg_sh):
    c = lax.axis_index("c")
    s = lax.axis_index("s")
    w = c * NS + s

    @pl.loop(0, rpt // L)
    def _(i):
      zbuf[pl.ds(i * L, L)] = jnp.zeros((L,), jnp.float32)

    pltpu.sync_copy(zbuf, deg_sh.at[pl.ds(s * rpt, rpt)])
    plsc.subcore_barrier()

    base = w * ew_per

    @pl.loop(0, ew_per // CH)
    def _(g):
      o = base + g * CH
      pltpu.sync_copy(dst_hbm.at[pl.ds(o, CH)], idx)
      pltpu.sync_copy(ew_hbm.at[pl.ds(o, CH)], val)
      pltpu.sync_copy(val, deg_sh.at[idx], add=True)

    plsc.subcore_barrier()
    pltpu.sync_copy(deg_sh.at[pl.ds(s * rpt, rpt)],
                    out_hbm.at[c, pl.ds(s * rpt, rpt)])

  return k(dst, ew)


CHM = 64


def _sc_msgpass(hp, src2d, dst2d, ew):
  n_nodes, d = hp.shape
  e = ew.shape[0]
  ew_per = e // NW
  gpw = ew_per // CHM
  rpt = n_nodes // NS

  @functools.partial(
      pl.kernel,
      out_type=jax.ShapeDtypeStruct((NC, n_nodes, d), jnp.float32),
      mesh=_mesh(),
      scratch_types=(
          pltpu.VMEM((gpw, CHM), jnp.int32),
          pltpu.VMEM((gpw, CHM), jnp.int32),
          pltpu.VMEM((ew_per,), jnp.float32),
          pltpu.VMEM((CHM, d), jnp.float32),
          pltpu.VMEM((CHM, d), jnp.float32),
          pltpu.VMEM_SHARED((n_nodes, d), jnp.float32),
          pltpu.SemaphoreType.DMA,
          pltpu.SemaphoreType.DMA,
          pltpu.SemaphoreType.DMA,
          pltpu.SemaphoreType.DMA,
      ),
      compiler_params=pltpu.CompilerParams(use_tc_tiling_on_sc=False,
                                           needs_layout_passes=False),
  )
  def k(hp_hbm, src_hbm, dst_hbm, ew_hbm, out_hbm,
        sidx, didx, wv, rows_a, rows_b, acc_sh, gsa, gsb, ssa, ssb):
    c = lax.axis_index("c")
    s = lax.axis_index("s")
    w = c * NS + s

    pltpu.sync_copy(src_hbm.at[pl.ds(w * gpw, gpw), :], sidx)
    pltpu.sync_copy(dst_hbm.at[pl.ds(w * gpw, gpw), :], didx)
    pltpu.sync_copy(ew_hbm.at[pl.ds(w * ew_per, ew_per)], wv)

    @pl.loop(0, CHM)
    def _(i):
      for j in range(d // L):
        rows_a[i, pl.ds(j * L, L)] = jnp.zeros((L,), jnp.float32)

    @pl.loop(0, rpt // CHM)
    def _(t):
      pltpu.sync_copy(rows_a, acc_sh.at[pl.ds(s * rpt + t * CHM, CHM), :])

    plsc.subcore_barrier()

    lanes = lax.iota(jnp.int32, L)

    def scale(rows, g):
      @pl.loop(0, CHM // L)
      def _(i):
        w16 = wv[pl.ds(g * CHM + i * L, L)]
        for k in range(L):
          wl = jnp.take_along_axis(w16, jnp.full((L,), k, jnp.int32),
                                   axis=0, mode="promise_in_bounds")
          row = i * L + k
          for j in range(d // L):
            rows[row, pl.ds(j * L, L)] = rows[row, pl.ds(j * L, L)] * wl

    def start_gather(rows, sem, g):
      pltpu.async_copy(hp_hbm.at[sidx.at[g]], rows, sem)

    def wait_gather(rows, sem):
      pltpu.make_async_copy(hp_hbm.at[sidx.at[0]], rows, sem).wait()

    def wait_scatter(rows, sem):
      pltpu.make_async_copy(rows, acc_sh.at[didx.at[0]], sem).wait()

    start_gather(rows_a, gsa, 0)
    start_gather(rows_b, gsb, 1)

    @pl.loop(0, gpw, step=2)
    def _(g):
      wait_gather(rows_a, gsa)
      scale(rows_a, g)
      pltpu.async_copy(rows_a, acc_sh.at[didx.at[g]], ssa, add=True)
      wait_gather(rows_b, gsb)
      scale(rows_b, g + 1)
      pltpu.async_copy(rows_b, acc_sh.at[didx.at[g + 1]], ssb, add=True)
      wait_scatter(rows_a, ssa)
      start_gather(rows_a, gsa, jnp.minimum(g + 2, gpw - 1))
      wait_scatter(rows_b, ssb)
      start_gather(rows_b, gsb, jnp.minimum(g + 3, gpw - 1))

    wait_gather(rows_a, gsa)
    wait_gather(rows_b, gsb)
    plsc.subcore_barrier()
    pltpu.sync_copy(acc_sh.at[pl.ds(s * rpt, rpt), :],
                    out_hbm.at[c, pl.ds(s * rpt, rpt), :])

  return k(hp, src2d, dst2d, ew)


def _tc_zeros(nn):
  blk = 524288
  grid = nn // blk

  def body(out_ref):
    out_ref[...] = jnp.zeros((blk,), jnp.float32)

  return pl.pallas_call(
      body,
      grid=(grid,),
      out_specs=pl.BlockSpec((blk,), lambda i: (i,)),
      out_shape=jax.ShapeDtypeStruct((nn,), jnp.float32),
  )()


def _sc_adj_scatter(src2d, dst2d, ew, adj_ref, n_nodes):
  e = ew.shape[0]
  e_per = e // NW
  gpt = e_per // CH

  @functools.partial(
      pl.kernel,
      out_type=(),
      mesh=_mesh(),
      scratch_types=(
          pltpu.VMEM((gpt, CH), jnp.int32),
          pltpu.VMEM((gpt, CH), jnp.int32),
          pltpu.VMEM((gpt, CH), jnp.int32),
          pltpu.VMEM((e_per,), jnp.float32),
          pltpu.SemaphoreType.DMA,
      ),
  )
  def k(src_hbm, dst_hbm, ew_hbm, out_hbm, sidx, didx, fidx, wv, sem):
    c = lax.axis_index("c")
    s = lax.axis_index("s")
    w = c * NS + s

    pltpu.sync_copy(src_hbm.at[pl.ds(w * gpt, gpt), :], sidx)
    pltpu.sync_copy(dst_hbm.at[pl.ds(w * gpt, gpt), :], didx)
    pltpu.sync_copy(ew_hbm.at[pl.ds(w * e_per, e_per)], wv)

    @pl.loop(0, gpt)
    def _(g):
      @pl.loop(0, CH // L)
      def _(j):
        sv = sidx[g, pl.ds(j * L, L)]
        dv = didx[g, pl.ds(j * L, L)]
        fidx[g, pl.ds(j * L, L)] = sv * n_nodes + dv

    @pl.loop(0, gpt // 16)
    def _(t):
      @pl.loop(0, 16)
      def _(u):
        g = t * 16 + u
        pltpu.async_copy(wv.at[pl.ds(g * CH, CH)], out_hbm.at[fidx.at[g]],
                         sem)

      @pl.loop(0, 16)
      def _(u):
        pltpu.make_async_copy(wv.at[pl.ds(0, CH)], out_hbm.at[fidx.at[0]],
                              sem).wait()

  return k(src2d, dst2d, ew, adj_ref)


def _tc_layer1(x, w1, b1, degp_t):
  n_nodes, d = x.shape
  r = 256
  grid = n_nodes // r

  def body(x_ref, w_ref, b_ref, deg_ref, hp_ref, dinv_ref):
    deg = jnp.sum(deg_ref[...], axis=1, keepdims=True) + 1.0
    dinv = jnp.where(deg > 0, lax.rsqrt(deg), 0.0)
    h = jnp.dot(x_ref[...], w_ref[...], preferred_element_type=jnp.float32)
    h = h + b_ref[...]
    hp_ref[...] = h * dinv
    dinv_ref[...] = dinv

  return pl.pallas_call(
      body,
      grid=(grid,),
      in_specs=[
          pl.BlockSpec((r, d), lambda i: (i, 0)),
          pl.BlockSpec((d, d), lambda i: (0, 0)),
          pl.BlockSpec((1, d), lambda i: (0, 0)),
          pl.BlockSpec((r, NC), lambda i: (i, 0)),
      ],
      out_specs=[
          pl.BlockSpec((r, d), lambda i: (i, 0)),
          pl.BlockSpec((r, 1), lambda i: (i, 0)),
      ],
      out_shape=[
          jax.ShapeDtypeStruct((n_nodes, d), jnp.float32),
          jax.ShapeDtypeStruct((n_nodes, 1), jnp.float32),
      ],
  )(x, w1, b1, degp_t)


def _tc_layer2(acc1, hp1, dinv, w2, b2):
  _, n_nodes, d = acc1.shape
  r = 256
  grid = n_nodes // r

  def body(acc_ref, hp_ref, dinv_ref, w_ref, b_ref, out_ref):
    dinv_blk = dinv_ref[...]
    h1 = dinv_blk * (acc_ref[0] + acc_ref[1] + hp_ref[...])
    x2 = jnp.maximum(h1, 0.0)
    h2 = jnp.dot(x2, w_ref[...], preferred_element_type=jnp.float32)
    h2 = h2 + b_ref[...]
    out_ref[...] = h2 * dinv_blk

  return pl.pallas_call(
      body,
      grid=(grid,),
      in_specs=[
          pl.BlockSpec((NC, r, d), lambda i: (0, i, 0)),
          pl.BlockSpec((r, d), lambda i: (i, 0)),
          pl.BlockSpec((r, 1), lambda i: (i, 0)),
          pl.BlockSpec((d, d), lambda i: (0, 0)),
          pl.BlockSpec((1, d), lambda i: (0, 0)),
      ],
      out_specs=pl.BlockSpec((r, d), lambda i: (i, 0)),
      out_shape=jax.ShapeDtypeStruct((n_nodes, d), jnp.float32),
  )(acc1, hp1, dinv, w2, b2)


def _tc_finalize(acc2, hp2, dinv, x):
  _, n_nodes, d = acc2.shape
  r = 256
  grid = n_nodes // r

  def body(acc_ref, hp_ref, dinv_ref, x_ref, z_ref, enc_ref):
    z = dinv_ref[...] * (acc_ref[0] + acc_ref[1] + hp_ref[...])
    z_ref[...] = z
    enc_ref[...] = z + x_ref[...]

  return pl.pallas_call(
      body,
      grid=(grid,),
      in_specs=[
          pl.BlockSpec((NC, r, d), lambda i: (0, i, 0)),
          pl.BlockSpec((r, d), lambda i: (i, 0)),
          pl.BlockSpec((r, 1), lambda i: (i, 0)),
          pl.BlockSpec((r, d), lambda i: (i, 0)),
      ],
      out_specs=[
          pl.BlockSpec((r, d), lambda i: (i, 0)),
          pl.BlockSpec((r, d), lambda i: (i, 0)),
      ],
      out_shape=[
          jax.ShapeDtypeStruct((n_nodes, d), jnp.float32),
          jax.ShapeDtypeStruct((n_nodes, d), jnp.float32),
      ],
  )(acc2, hp2, dinv, x)


def _tc_decoder(z, adj_flat):
  n_nodes, d = z.shape
  rb = 128
  grid = n_nodes // rb

  def body(z_ref, adj_ref, out_ref):
    i = pl.program_id(0)
    zr = z_ref[pl.ds(i * rb, rb), :]
    s = lax.dot_general(zr, z_ref[...],
                        dimension_numbers=(((1,), (1,)), ((), ())),
                        preferred_element_type=jnp.float32)
    adj = adj_ref[...].reshape(rb, n_nodes)
    out_ref[...] = jax.nn.sigmoid(s + adj)

  return pl.pallas_call(
      body,
      grid=(grid,),
      in_specs=[
          pl.BlockSpec((n_nodes, d), lambda i: (0, 0)),
          pl.BlockSpec((rb * n_nodes,), lambda i: (i,)),
      ],
      out_specs=pl.BlockSpec((rb, n_nodes), lambda i: (i, 0)),
      out_shape=jax.ShapeDtypeStruct((n_nodes, n_nodes), jnp.float32),
  )(z, adj_flat)


def kernel(node_features, edge_list, edge_attr, batch, W1, b1, W2, b2):
  n_nodes, d = node_features.shape
  src = edge_list[0]
  dst = edge_list[1]
  ew = edge_attr[:, 0]
  src2d = src.reshape(-1, CH)
  dst2d = dst.reshape(-1, CH)
  src2m = src.reshape(-1, CHM)
  dst2m = dst.reshape(-1, CHM)

  degp = _sc_degree(dst, ew, n_nodes)
  adj_ref = jax.new_ref(_tc_zeros(n_nodes * n_nodes))
  _sc_adj_scatter(src2d, dst2d, ew, adj_ref, n_nodes)
  adj_flat = adj_ref[...]

  hp1, dinv = _tc_layer1(node_features, W1, b1.reshape(1, d),
                         degp.T)
  acc1 = _sc_msgpass(hp1, src2m, dst2m, ew)
  hp2 = _tc_layer2(acc1, hp1, dinv, W2, b2.reshape(1, d))
  acc2 = _sc_msgpass(hp2, src2m, dst2m, ew)
  z, encoded = _tc_finalize(acc2, hp2, dinv, node_features)
  probs = _tc_decoder(z, adj_flat)
  return (encoded, edge_list, probs)


# speedup vs baseline: 47.7194x; 1.1816x over previous
import functools

import jax
import jax.numpy as jnp
from jax import lax
from jax.experimental import pallas as pl
from jax.experimental.pallas import tpu as pltpu
from jax.experimental.pallas import tpu_sc as plsc

NC = 2
NS = 16
L = 16
NW = NC * NS

CH = 128

_SC_MESH = dict(core_axis_name="c", subcore_axis_name="s")


def _mesh():
  return plsc.VectorSubcoreMesh(**_SC_MESH)


def _sc_degree(dst, ew, n_nodes):
  e = dst.shape[0]
  ew_per = e // NW
  rpt = n_nodes // NS

  @functools.partial(
      pl.kernel,
      out_type=jax.ShapeDtypeStruct((NC, n_nodes), jnp.float32),
      mesh=_mesh(),
      scratch_types=(
          pltpu.VMEM((CH,), jnp.int32),
          pltpu.VMEM((CH,), jnp.float32),
          pltpu.VMEM((rpt,), jnp.float32),
          pltpu.VMEM_SHARED((n_nodes,), jnp.float32),
      ),
  )
  def k(dst_hbm, ew_hbm, out_hbm, idx, val, zbuf, deg_sh):
    c = lax.axis_index("c")
    s = lax.axis_index("s")
    w = c * NS + s

    @pl.loop(0, rpt // L)
    def _(i):
      zbuf[pl.ds(i * L, L)] = jnp.zeros((L,), jnp.float32)

    pltpu.sync_copy(zbuf, deg_sh.at[pl.ds(s * rpt, rpt)])
    plsc.subcore_barrier()

    base = w * ew_per

    @pl.loop(0, ew_per // CH)
    def _(g):
      o = base + g * CH
      pltpu.sync_copy(dst_hbm.at[pl.ds(o, CH)], idx)
      pltpu.sync_copy(ew_hbm.at[pl.ds(o, CH)], val)
      pltpu.sync_copy(val, deg_sh.at[idx], add=True)

    plsc.subcore_barrier()
    pltpu.sync_copy(deg_sh.at[pl.ds(s * rpt, rpt)],
                    out_hbm.at[c, pl.ds(s * rpt, rpt)])

  return k(dst, ew)


CHM = 64


def _sc_msgpass(hp, src2d, dst2d, ew, adj_ref=None):
  n_nodes, d = hp.shape
  e = ew.shape[0]
  ew_per = e // NW
  gpw = ew_per // CHM
  rpt = n_nodes // NS
  with_adj = adj_ref is not None

  scratch = [
      pltpu.VMEM((gpw, CHM), jnp.int32),
      pltpu.VMEM((gpw, CHM), jnp.int32),
      pltpu.VMEM((ew_per,), jnp.float32),
      pltpu.VMEM((CHM, d), jnp.float32),
      pltpu.VMEM((CHM, d), jnp.float32),
      pltpu.VMEM_SHARED((n_nodes, d), jnp.float32),
      pltpu.SemaphoreType.DMA,
      pltpu.SemaphoreType.DMA,
      pltpu.SemaphoreType.DMA,
      pltpu.SemaphoreType.DMA,
  ]
  if with_adj:
    scratch.append(pltpu.VMEM((gpw, CHM), jnp.int32))
    scratch.append(pltpu.SemaphoreType.DMA)

  @functools.partial(
      pl.kernel,
      out_type=jax.ShapeDtypeStruct((NC, n_nodes, d), jnp.float32),
      mesh=_mesh(),
      scratch_types=tuple(scratch),
      compiler_params=pltpu.CompilerParams(use_tc_tiling_on_sc=False,
                                           needs_layout_passes=False),
  )
  def k(hp_hbm, src_hbm, dst_hbm, ew_hbm, *rest):
    if with_adj:
      (adj_hbm, out_hbm, sidx, didx, wv, rows_a, rows_b, acc_sh,
       gsa, gsb, ssa, ssb, fidx, sadj) = rest
    else:
      (out_hbm, sidx, didx, wv, rows_a, rows_b, acc_sh,
       gsa, gsb, ssa, ssb) = rest
    c = lax.axis_index("c")
    s = lax.axis_index("s")
    w = c * NS + s

    pltpu.sync_copy(src_hbm.at[pl.ds(w * gpw, gpw), :], sidx)
    pltpu.sync_copy(dst_hbm.at[pl.ds(w * gpw, gpw), :], didx)
    pltpu.sync_copy(ew_hbm.at[pl.ds(w * ew_per, ew_per)], wv)

    @pl.loop(0, CHM)
    def _(i):
      for j in range(d // L):
        rows_a[i, pl.ds(j * L, L)] = jnp.zeros((L,), jnp.float32)

    @pl.loop(0, rpt // CHM)
    def _(t):
      pltpu.sync_copy(rows_a, acc_sh.at[pl.ds(s * rpt + t * CHM, CHM), :])

    plsc.subcore_barrier()

    def scale(rows, g):
      @pl.loop(0, CHM // L)
      def _(i):
        w16 = wv[pl.ds(g * CHM + i * L, L)]
        for k in range(L):
          wl = jnp.take_along_axis(w16, jnp.full((L,), k, jnp.int32),
                                   axis=0, mode="promise_in_bounds")
          row = i * L + k
          for j in range(d // L):
            rows[row, pl.ds(j * L, L)] = rows[row, pl.ds(j * L, L)] * wl

    def start_gather(rows, sem, g):
      pltpu.async_copy(hp_hbm.at[sidx.at[g]], rows, sem)

    def wait_gather(rows, sem):
      pltpu.make_async_copy(hp_hbm.at[sidx.at[0]], rows, sem).wait()

    def wait_scatter(rows, sem):
      pltpu.make_async_copy(rows, acc_sh.at[didx.at[0]], sem).wait()

    if with_adj:
      @pl.loop(0, gpw)
      def _(g):
        @pl.loop(0, CHM // L)
        def _(j):
          sv = sidx[g, pl.ds(j * L, L)]
          dv = didx[g, pl.ds(j * L, L)]
          fidx[g, pl.ds(j * L, L)] = sv * n_nodes + dv

    def fire_adj(g):
      if with_adj:
        pltpu.async_copy(wv.at[pl.ds(g * CHM, CHM)], adj_hbm.at[fidx.at[g]],
                         sadj)

    start_gather(rows_a, gsa, 0)
    start_gather(rows_b, gsb, 1)

    @pl.loop(0, gpw, step=2)
    def _(g):
      fire_adj(g)
      wait_gather(rows_a, gsa)
      scale(rows_a, g)
      pltpu.async_copy(rows_a, acc_sh.at[didx.at[g]], ssa, add=True)
      fire_adj(g + 1)
      wait_gather(rows_b, gsb)
      scale(rows_b, g + 1)
      pltpu.async_copy(rows_b, acc_sh.at[didx.at[g + 1]], ssb, add=True)
      wait_scatter(rows_a, ssa)
      start_gather(rows_a, gsa, jnp.minimum(g + 2, gpw - 1))
      wait_scatter(rows_b, ssb)
      start_gather(rows_b, gsb, jnp.minimum(g + 3, gpw - 1))

    wait_gather(rows_a, gsa)
    wait_gather(rows_b, gsb)
    if with_adj:
      @pl.loop(0, gpw)
      def _(g):
        pltpu.make_async_copy(wv.at[pl.ds(0, CHM)], adj_hbm.at[fidx.at[0]],
                              sadj).wait()
    plsc.subcore_barrier()
    pltpu.sync_copy(acc_sh.at[pl.ds(s * rpt, rpt), :],
                    out_hbm.at[c, pl.ds(s * rpt, rpt), :])

  if with_adj:
    return k(hp, src2d, dst2d, ew, adj_ref)
  return k(hp, src2d, dst2d, ew)


def _tc_zeros(nn):
  blk = 524288
  grid = nn // blk

  def body(out_ref):
    out_ref[...] = jnp.zeros((blk,), jnp.float32)

  return pl.pallas_call(
      body,
      grid=(grid,),
      out_specs=pl.BlockSpec((blk,), lambda i: (i,)),
      out_shape=jax.ShapeDtypeStruct((nn,), jnp.float32),
  )()


def _tc_layer1(x, w1, b1, degp_t):
  n_nodes, d = x.shape
  r = 256
  grid = n_nodes // r

  def body(x_ref, w_ref, b_ref, deg_ref, hp_ref, dinv_ref):
    deg = jnp.sum(deg_ref[...], axis=1, keepdims=True) + 1.0
    dinv = jnp.where(deg > 0, lax.rsqrt(deg), 0.0)
    h = jnp.dot(x_ref[...], w_ref[...], preferred_element_type=jnp.float32)
    h = h + b_ref[...]
    hp_ref[...] = h * dinv
    dinv_ref[...] = dinv

  return pl.pallas_call(
      body,
      grid=(grid,),
      in_specs=[
          pl.BlockSpec((r, d), lambda i: (i, 0)),
          pl.BlockSpec((d, d), lambda i: (0, 0)),
          pl.BlockSpec((1, d), lambda i: (0, 0)),
          pl.BlockSpec((r, NC), lambda i: (i, 0)),
      ],
      out_specs=[
          pl.BlockSpec((r, d), lambda i: (i, 0)),
          pl.BlockSpec((r, 1), lambda i: (i, 0)),
      ],
      out_shape=[
          jax.ShapeDtypeStruct((n_nodes, d), jnp.float32),
          jax.ShapeDtypeStruct((n_nodes, 1), jnp.float32),
      ],
  )(x, w1, b1, degp_t)


def _tc_layer2(acc1, hp1, dinv, w2, b2):
  _, n_nodes, d = acc1.shape
  r = 256
  grid = n_nodes // r

  def body(acc_ref, hp_ref, dinv_ref, w_ref, b_ref, out_ref):
    dinv_blk = dinv_ref[...]
    h1 = dinv_blk * (acc_ref[0] + acc_ref[1] + hp_ref[...])
    x2 = jnp.maximum(h1, 0.0)
    h2 = jnp.dot(x2, w_ref[...], preferred_element_type=jnp.float32)
    h2 = h2 + b_ref[...]
    out_ref[...] = h2 * dinv_blk

  return pl.pallas_call(
      body,
      grid=(grid,),
      in_specs=[
          pl.BlockSpec((NC, r, d), lambda i: (0, i, 0)),
          pl.BlockSpec((r, d), lambda i: (i, 0)),
          pl.BlockSpec((r, 1), lambda i: (i, 0)),
          pl.BlockSpec((d, d), lambda i: (0, 0)),
          pl.BlockSpec((1, d), lambda i: (0, 0)),
      ],
      out_specs=pl.BlockSpec((r, d), lambda i: (i, 0)),
      out_shape=jax.ShapeDtypeStruct((n_nodes, d), jnp.float32),
  )(acc1, hp1, dinv, w2, b2)


def _tc_finalize(acc2, hp2, dinv, x):
  _, n_nodes, d = acc2.shape
  r = 256
  grid = n_nodes // r

  def body(acc_ref, hp_ref, dinv_ref, x_ref, z_ref, enc_ref):
    z = dinv_ref[...] * (acc_ref[0] + acc_ref[1] + hp_ref[...])
    z_ref[...] = z
    enc_ref[...] = z + x_ref[...]

  return pl.pallas_call(
      body,
      grid=(grid,),
      in_specs=[
          pl.BlockSpec((NC, r, d), lambda i: (0, i, 0)),
          pl.BlockSpec((r, d), lambda i: (i, 0)),
          pl.BlockSpec((r, 1), lambda i: (i, 0)),
          pl.BlockSpec((r, d), lambda i: (i, 0)),
      ],
      out_specs=[
          pl.BlockSpec((r, d), lambda i: (i, 0)),
          pl.BlockSpec((r, d), lambda i: (i, 0)),
      ],
      out_shape=[
          jax.ShapeDtypeStruct((n_nodes, d), jnp.float32),
          jax.ShapeDtypeStruct((n_nodes, d), jnp.float32),
      ],
  )(acc2, hp2, dinv, x)


def _tc_decoder(z, adj_flat):
  n_nodes, d = z.shape
  rb = 128
  grid = n_nodes // rb

  def body(z_ref, adj_ref, out_ref):
    i = pl.program_id(0)
    zr = z_ref[pl.ds(i * rb, rb), :]
    s = lax.dot_general(zr, z_ref[...],
                        dimension_numbers=(((1,), (1,)), ((), ())),
                        preferred_element_type=jnp.float32)
    adj = adj_ref[...].reshape(rb, n_nodes)
    out_ref[...] = jax.nn.sigmoid(s + adj)

  return pl.pallas_call(
      body,
      grid=(grid,),
      in_specs=[
          pl.BlockSpec((n_nodes, d), lambda i: (0, 0)),
          pl.BlockSpec((rb * n_nodes,), lambda i: (i,)),
      ],
      out_specs=pl.BlockSpec((rb, n_nodes), lambda i: (i, 0)),
      out_shape=jax.ShapeDtypeStruct((n_nodes, n_nodes), jnp.float32),
  )(z, adj_flat)


def kernel(node_features, edge_list, edge_attr, batch, W1, b1, W2, b2):
  n_nodes, d = node_features.shape
  src = edge_list[0]
  dst = edge_list[1]
  ew = edge_attr[:, 0]
  src2m = src.reshape(-1, CHM)
  dst2m = dst.reshape(-1, CHM)

  degp = _sc_degree(dst, ew, n_nodes)
  adj_ref = jax.new_ref(_tc_zeros(n_nodes * n_nodes))

  hp1, dinv = _tc_layer1(node_features, W1, b1.reshape(1, d),
                         degp.T)
  acc1 = _sc_msgpass(hp1, src2m, dst2m, ew, adj_ref)
  adj_flat = adj_ref[...]
  hp2 = _tc_layer2(acc1, hp1, dinv, W2, b2.reshape(1, d))
  acc2 = _sc_msgpass(hp2, src2m, dst2m, ew)
  z, encoded = _tc_finalize(acc2, hp2, dinv, node_features)
  probs = _tc_decoder(z, adj_flat)
  return (encoded, edge_list, probs)


# speedup vs baseline: 49.5898x; 1.0392x over previous
import functools

import jax
import jax.numpy as jnp
from jax import lax
from jax.experimental import pallas as pl
from jax.experimental.pallas import tpu as pltpu
from jax.experimental.pallas import tpu_sc as plsc

NC = 2
NS = 16
L = 16
NW = NC * NS

CH = 128

_SC_MESH = dict(core_axis_name="c", subcore_axis_name="s")


def _mesh():
  return plsc.VectorSubcoreMesh(**_SC_MESH)


def _sc_degree(dst2d, ew, n_nodes):
  e = ew.shape[0]
  ew_per = e // NW
  gpd = ew_per // CH
  rpt = n_nodes // NS

  @functools.partial(
      pl.kernel,
      out_type=jax.ShapeDtypeStruct((NC, n_nodes), jnp.float32),
      mesh=_mesh(),
      scratch_types=(
          pltpu.VMEM((gpd, CH), jnp.int32),
          pltpu.VMEM((ew_per,), jnp.float32),
          pltpu.VMEM((rpt,), jnp.float32),
          pltpu.VMEM_SHARED((n_nodes,), jnp.float32),
          pltpu.SemaphoreType.DMA,
      ),
  )
  def k(dst_hbm, ew_hbm, out_hbm, idx, val, zbuf, deg_sh, sem):
    c = lax.axis_index("c")
    s = lax.axis_index("s")
    w = c * NS + s

    pltpu.sync_copy(dst_hbm.at[pl.ds(w * gpd, gpd), :], idx)
    pltpu.sync_copy(ew_hbm.at[pl.ds(w * ew_per, ew_per)], val)

    @pl.loop(0, rpt // L)
    def _(i):
      zbuf[pl.ds(i * L, L)] = jnp.zeros((L,), jnp.float32)

    pltpu.sync_copy(zbuf, deg_sh.at[pl.ds(s * rpt, rpt)])
    plsc.subcore_barrier()

    @pl.loop(0, gpd)
    def _(g):
      pltpu.async_copy(val.at[pl.ds(g * CH, CH)], deg_sh.at[idx.at[g]],
                       sem, add=True)

    @pl.loop(0, gpd)
    def _(g):
      pltpu.make_async_copy(val.at[pl.ds(0, CH)], deg_sh.at[idx.at[0]],
                            sem).wait()

    plsc.subcore_barrier()
    pltpu.sync_copy(deg_sh.at[pl.ds(s * rpt, rpt)],
                    out_hbm.at[c, pl.ds(s * rpt, rpt)])

  return k(dst2d, ew)


CHM = 64


def _sc_msgpass(hp, src2d, dst2d, ew, adj_ref=None):
  n_nodes, d = hp.shape
  e = ew.shape[0]
  ew_per = e // NW
  gpw = ew_per // CHM
  rpt = n_nodes // NS
  with_adj = adj_ref is not None

  scratch = [
      pltpu.VMEM((gpw, CHM), jnp.int32),
      pltpu.VMEM((gpw, CHM), jnp.int32),
      pltpu.VMEM((ew_per,), jnp.float32),
      pltpu.VMEM((CHM, d), jnp.float32),
      pltpu.VMEM((CHM, d), jnp.float32),
      pltpu.VMEM_SHARED((n_nodes, d), jnp.float32),
      pltpu.SemaphoreType.DMA,
      pltpu.SemaphoreType.DMA,
      pltpu.SemaphoreType.DMA,
      pltpu.SemaphoreType.DMA,
  ]
  if with_adj:
    scratch.append(pltpu.VMEM((gpw, CHM), jnp.int32))
    scratch.append(pltpu.SemaphoreType.DMA)

  @functools.partial(
      pl.kernel,
      out_type=jax.ShapeDtypeStruct((NC, n_nodes, d), jnp.float32),
      mesh=_mesh(),
      scratch_types=tuple(scratch),
      compiler_params=pltpu.CompilerParams(use_tc_tiling_on_sc=False,
                                           needs_layout_passes=False),
  )
  def k(hp_hbm, src_hbm, dst_hbm, ew_hbm, *rest):
    if with_adj:
      (adj_hbm, out_hbm, sidx, didx, wv, rows_a, rows_b, acc_sh,
       gsa, gsb, ssa, ssb, fidx, sadj) = rest
    else:
      (out_hbm, sidx, didx, wv, rows_a, rows_b, acc_sh,
       gsa, gsb, ssa, ssb) = rest
    c = lax.axis_index("c")
    s = lax.axis_index("s")
    w = c * NS + s

    pltpu.sync_copy(src_hbm.at[pl.ds(w * gpw, gpw), :], sidx)
    pltpu.sync_copy(dst_hbm.at[pl.ds(w * gpw, gpw), :], didx)
    pltpu.sync_copy(ew_hbm.at[pl.ds(w * ew_per, ew_per)], wv)

    @pl.loop(0, CHM)
    def _(i):
      for j in range(d // L):
        rows_a[i, pl.ds(j * L, L)] = jnp.zeros((L,), jnp.float32)

    @pl.loop(0, rpt // CHM)
    def _(t):
      pltpu.sync_copy(rows_a, acc_sh.at[pl.ds(s * rpt + t * CHM, CHM), :])

    plsc.subcore_barrier()

    def scale(rows, g):
      @pl.loop(0, CHM // L)
      def _(i):
        w16 = wv[pl.ds(g * CHM + i * L, L)]
        for k in range(L):
          wl = jnp.take_along_axis(w16, jnp.full((L,), k, jnp.int32),
                                   axis=0, mode="promise_in_bounds")
          row = i * L + k
          for j in range(d // L):
            rows[row, pl.ds(j * L, L)] = rows[row, pl.ds(j * L, L)] * wl

    def start_gather(rows, sem, g):
      pltpu.async_copy(hp_hbm.at[sidx.at[g]], rows, sem)

    def wait_gather(rows, sem):
      pltpu.make_async_copy(hp_hbm.at[sidx.at[0]], rows, sem).wait()

    def wait_scatter(rows, sem):
      pltpu.make_async_copy(rows, acc_sh.at[didx.at[0]], sem).wait()

    if with_adj:
      @pl.loop(0, gpw)
      def _(g):
        @pl.loop(0, CHM // L)
        def _(j):
          sv = sidx[g, pl.ds(j * L, L)]
          dv = didx[g, pl.ds(j * L, L)]
          fidx[g, pl.ds(j * L, L)] = sv * n_nodes + dv

    def fire_adj(g):
      if with_adj:
        pltpu.async_copy(wv.at[pl.ds(g * CHM, CHM)], adj_hbm.at[fidx.at[g]],
                         sadj)

    start_gather(rows_a, gsa, 0)
    start_gather(rows_b, gsb, 1)

    @pl.loop(0, gpw, step=2)
    def _(g):
      fire_adj(g)
      wait_gather(rows_a, gsa)
      scale(rows_a, g)
      pltpu.async_copy(rows_a, acc_sh.at[didx.at[g]], ssa, add=True)
      fire_adj(g + 1)
      wait_gather(rows_b, gsb)
      scale(rows_b, g + 1)
      pltpu.async_copy(rows_b, acc_sh.at[didx.at[g + 1]], ssb, add=True)
      wait_scatter(rows_a, ssa)
      start_gather(rows_a, gsa, jnp.minimum(g + 2, gpw - 1))
      wait_scatter(rows_b, ssb)
      start_gather(rows_b, gsb, jnp.minimum(g + 3, gpw - 1))

    wait_gather(rows_a, gsa)
    wait_gather(rows_b, gsb)
    if with_adj:
      @pl.loop(0, gpw)
      def _(g):
        pltpu.make_async_copy(wv.at[pl.ds(0, CHM)], adj_hbm.at[fidx.at[0]],
                              sadj).wait()
    plsc.subcore_barrier()
    pltpu.sync_copy(acc_sh.at[pl.ds(s * rpt, rpt), :],
                    out_hbm.at[c, pl.ds(s * rpt, rpt), :])

  if with_adj:
    return k(hp, src2d, dst2d, ew, adj_ref)
  return k(hp, src2d, dst2d, ew)


def _tc_zeros(nn):
  blk = 524288
  grid = nn // blk

  def body(out_ref):
    out_ref[...] = jnp.zeros((blk,), jnp.float32)

  return pl.pallas_call(
      body,
      grid=(grid,),
      out_specs=pl.BlockSpec((blk,), lambda i: (i,)),
      out_shape=jax.ShapeDtypeStruct((nn,), jnp.float32),
  )()


def _tc_layer1(x, w1, b1, degp_t):
  n_nodes, d = x.shape
  r = 256
  grid = n_nodes // r

  def body(x_ref, w_ref, b_ref, deg_ref, hp_ref, dinv_ref):
    deg = jnp.sum(deg_ref[...], axis=1, keepdims=True) + 1.0
    dinv = jnp.where(deg > 0, lax.rsqrt(deg), 0.0)
    h = jnp.dot(x_ref[...], w_ref[...], preferred_element_type=jnp.float32)
    h = h + b_ref[...]
    hp_ref[...] = h * dinv
    dinv_ref[...] = dinv

  return pl.pallas_call(
      body,
      grid=(grid,),
      in_specs=[
          pl.BlockSpec((r, d), lambda i: (i, 0)),
          pl.BlockSpec((d, d), lambda i: (0, 0)),
          pl.BlockSpec((1, d), lambda i: (0, 0)),
          pl.BlockSpec((r, NC), lambda i: (i, 0)),
      ],
      out_specs=[
          pl.BlockSpec((r, d), lambda i: (i, 0)),
          pl.BlockSpec((r, 1), lambda i: (i, 0)),
      ],
      out_shape=[
          jax.ShapeDtypeStruct((n_nodes, d), jnp.float32),
          jax.ShapeDtypeStruct((n_nodes, 1), jnp.float32),
      ],
  )(x, w1, b1, degp_t)


def _tc_layer2(acc1, hp1, dinv, w2, b2):
  _, n_nodes, d = acc1.shape
  r = 256
  grid = n_nodes // r

  def body(acc_ref, hp_ref, dinv_ref, w_ref, b_ref, out_ref):
    dinv_blk = dinv_ref[...]
    h1 = dinv_blk * (acc_ref[0] + acc_ref[1] + hp_ref[...])
    x2 = jnp.maximum(h1, 0.0)
    h2 = jnp.dot(x2, w_ref[...], preferred_element_type=jnp.float32)
    h2 = h2 + b_ref[...]
    out_ref[...] = h2 * dinv_blk

  return pl.pallas_call(
      body,
      grid=(grid,),
      in_specs=[
          pl.BlockSpec((NC, r, d), lambda i: (0, i, 0)),
          pl.BlockSpec((r, d), lambda i: (i, 0)),
          pl.BlockSpec((r, 1), lambda i: (i, 0)),
          pl.BlockSpec((d, d), lambda i: (0, 0)),
          pl.BlockSpec((1, d), lambda i: (0, 0)),
      ],
      out_specs=pl.BlockSpec((r, d), lambda i: (i, 0)),
      out_shape=jax.ShapeDtypeStruct((n_nodes, d), jnp.float32),
  )(acc1, hp1, dinv, w2, b2)


def _tc_finalize(acc2, hp2, dinv, x):
  _, n_nodes, d = acc2.shape
  r = 256
  grid = n_nodes // r

  def body(acc_ref, hp_ref, dinv_ref, x_ref, z_ref, enc_ref):
    z = dinv_ref[...] * (acc_ref[0] + acc_ref[1] + hp_ref[...])
    z_ref[...] = z
    enc_ref[...] = z + x_ref[...]

  return pl.pallas_call(
      body,
      grid=(grid,),
      in_specs=[
          pl.BlockSpec((NC, r, d), lambda i: (0, i, 0)),
          pl.BlockSpec((r, d), lambda i: (i, 0)),
          pl.BlockSpec((r, 1), lambda i: (i, 0)),
          pl.BlockSpec((r, d), lambda i: (i, 0)),
      ],
      out_specs=[
          pl.BlockSpec((r, d), lambda i: (i, 0)),
          pl.BlockSpec((r, d), lambda i: (i, 0)),
      ],
      out_shape=[
          jax.ShapeDtypeStruct((n_nodes, d), jnp.float32),
          jax.ShapeDtypeStruct((n_nodes, d), jnp.float32),
      ],
  )(acc2, hp2, dinv, x)


def _tc_decoder(z, adj_flat):
  n_nodes, d = z.shape
  rb = 128
  grid = n_nodes // rb

  def body(z_ref, adj_ref, out_ref):
    i = pl.program_id(0)
    zr = z_ref[pl.ds(i * rb, rb), :]
    s = lax.dot_general(zr, z_ref[...],
                        dimension_numbers=(((1,), (1,)), ((), ())),
                        preferred_element_type=jnp.float32)
    adj = adj_ref[...].reshape(rb, n_nodes)
    out_ref[...] = jax.nn.sigmoid(s + adj)

  return pl.pallas_call(
      body,
      grid=(grid,),
      in_specs=[
          pl.BlockSpec((n_nodes, d), lambda i: (0, 0)),
          pl.BlockSpec((rb * n_nodes,), lambda i: (i,)),
      ],
      out_specs=pl.BlockSpec((rb, n_nodes), lambda i: (i, 0)),
      out_shape=jax.ShapeDtypeStruct((n_nodes, n_nodes), jnp.float32),
  )(z, adj_flat)


def kernel(node_features, edge_list, edge_attr, batch, W1, b1, W2, b2):
  n_nodes, d = node_features.shape
  src = edge_list[0]
  dst = edge_list[1]
  ew = edge_attr[:, 0]
  src2m = src.reshape(-1, CHM)
  dst2m = dst.reshape(-1, CHM)
  dst2d = dst.reshape(-1, CH)

  degp = _sc_degree(dst2d, ew, n_nodes)
  adj_ref = jax.new_ref(_tc_zeros(n_nodes * n_nodes))

  hp1, dinv = _tc_layer1(node_features, W1, b1.reshape(1, d),
                         degp.T)
  acc1 = _sc_msgpass(hp1, src2m, dst2m, ew, adj_ref)
  adj_flat = adj_ref[...]
  hp2 = _tc_layer2(acc1, hp1, dinv, W2, b2.reshape(1, d))
  acc2 = _sc_msgpass(hp2, src2m, dst2m, ew)
  z, encoded = _tc_finalize(acc2, hp2, dinv, node_features)
  probs = _tc_decoder(z, adj_flat)
  return (encoded, edge_list, probs)


# speedup vs baseline: 50.7868x; 1.0241x over previous
import functools

import jax
import jax.numpy as jnp
from jax import lax
from jax.experimental import pallas as pl
from jax.experimental.pallas import tpu as pltpu
from jax.experimental.pallas import tpu_sc as plsc

NC = 2
NS = 16
L = 16
NW = NC * NS

CH = 128

_SC_MESH = dict(core_axis_name="c", subcore_axis_name="s")


def _mesh():
  return plsc.VectorSubcoreMesh(**_SC_MESH)


def _sc_degree(dst2d, ew, n_nodes):
  e = ew.shape[0]
  ew_per = e // NW
  gpd = ew_per // CH
  rpt = n_nodes // NS

  @functools.partial(
      pl.kernel,
      out_type=jax.ShapeDtypeStruct((NC, n_nodes), jnp.float32),
      mesh=_mesh(),
      scratch_types=(
          pltpu.VMEM((gpd, CH), jnp.int32),
          pltpu.VMEM((ew_per,), jnp.float32),
          pltpu.VMEM((rpt,), jnp.float32),
          pltpu.VMEM_SHARED((n_nodes,), jnp.float32),
          pltpu.SemaphoreType.DMA,
      ),
  )
  def k(dst_hbm, ew_hbm, out_hbm, idx, val, zbuf, deg_sh, sem):
    c = lax.axis_index("c")
    s = lax.axis_index("s")
    w = c * NS + s

    pltpu.sync_copy(dst_hbm.at[pl.ds(w * gpd, gpd), :], idx)
    pltpu.sync_copy(ew_hbm.at[pl.ds(w * ew_per, ew_per)], val)

    @pl.loop(0, rpt // L)
    def _(i):
      zbuf[pl.ds(i * L, L)] = jnp.zeros((L,), jnp.float32)

    pltpu.sync_copy(zbuf, deg_sh.at[pl.ds(s * rpt, rpt)])
    plsc.subcore_barrier()

    @pl.loop(0, gpd)
    def _(g):
      pltpu.async_copy(val.at[pl.ds(g * CH, CH)], deg_sh.at[idx.at[g]],
                       sem, add=True)

    @pl.loop(0, gpd)
    def _(g):
      pltpu.make_async_copy(val.at[pl.ds(0, CH)], deg_sh.at[idx.at[0]],
                            sem).wait()

    plsc.subcore_barrier()
    pltpu.sync_copy(deg_sh.at[pl.ds(s * rpt, rpt)],
                    out_hbm.at[c, pl.ds(s * rpt, rpt)])

  return k(dst2d, ew)


CHM = 32
NBUF = 4


def _sc_msgpass(hp, src2d, dst2d, ew, adj_ref=None):
  n_nodes, d = hp.shape
  e = ew.shape[0]
  ew_per = e // NW
  gpw = ew_per // CHM
  rpt = n_nodes // NS
  with_adj = adj_ref is not None

  scratch = (
      [pltpu.VMEM((gpw, CHM), jnp.int32),
       pltpu.VMEM((gpw, CHM), jnp.int32),
       pltpu.VMEM((ew_per,), jnp.float32)]
      + [pltpu.VMEM((CHM, d), jnp.float32)] * NBUF
      + [pltpu.VMEM_SHARED((n_nodes, d), jnp.float32)]
      + [pltpu.SemaphoreType.DMA] * (2 * NBUF)
  )
  if with_adj:
    scratch.append(pltpu.VMEM((gpw, CHM), jnp.int32))
    scratch.append(pltpu.SemaphoreType.DMA)

  @functools.partial(
      pl.kernel,
      out_type=jax.ShapeDtypeStruct((NC, n_nodes, d), jnp.float32),
      mesh=_mesh(),
      scratch_types=tuple(scratch),
      compiler_params=pltpu.CompilerParams(use_tc_tiling_on_sc=False,
                                           needs_layout_passes=False),
  )
  def k(hp_hbm, src_hbm, dst_hbm, ew_hbm, *rest):
    if with_adj:
      adj_hbm = rest[0]
      rest = rest[1:]
    out_hbm, sidx, didx, wv = rest[:4]
    bufs = rest[4:4 + NBUF]
    acc_sh = rest[4 + NBUF]
    gsems = rest[5 + NBUF:5 + 2 * NBUF]
    ssems = rest[5 + 2 * NBUF:5 + 3 * NBUF]
    if with_adj:
      fidx, sadj = rest[5 + 3 * NBUF:]
    rows_a = bufs[0]
    c = lax.axis_index("c")
    s = lax.axis_index("s")
    w = c * NS + s

    pltpu.sync_copy(src_hbm.at[pl.ds(w * gpw, gpw), :], sidx)
    pltpu.sync_copy(dst_hbm.at[pl.ds(w * gpw, gpw), :], didx)
    pltpu.sync_copy(ew_hbm.at[pl.ds(w * ew_per, ew_per)], wv)

    @pl.loop(0, CHM)
    def _(i):
      for j in range(d // L):
        rows_a[i, pl.ds(j * L, L)] = jnp.zeros((L,), jnp.float32)

    @pl.loop(0, rpt // CHM)
    def _(t):
      pltpu.sync_copy(rows_a, acc_sh.at[pl.ds(s * rpt + t * CHM, CHM), :])

    plsc.subcore_barrier()

    def scale(rows, g):
      @pl.loop(0, CHM // L)
      def _(i):
        w16 = wv[pl.ds(g * CHM + i * L, L)]
        for k in range(L):
          wl = jnp.take_along_axis(w16, jnp.full((L,), k, jnp.int32),
                                   axis=0, mode="promise_in_bounds")
          row = i * L + k
          for j in range(d // L):
            rows[row, pl.ds(j * L, L)] = rows[row, pl.ds(j * L, L)] * wl

    def start_gather(rows, sem, g):
      pltpu.async_copy(hp_hbm.at[sidx.at[g]], rows, sem)

    def wait_gather(rows, sem):
      pltpu.make_async_copy(hp_hbm.at[sidx.at[0]], rows, sem).wait()

    def wait_scatter(rows, sem):
      pltpu.make_async_copy(rows, acc_sh.at[didx.at[0]], sem).wait()

    if with_adj:
      @pl.loop(0, gpw)
      def _(g):
        @pl.loop(0, CHM // L)
        def _(j):
          sv = sidx[g, pl.ds(j * L, L)]
          dv = didx[g, pl.ds(j * L, L)]
          fidx[g, pl.ds(j * L, L)] = sv * n_nodes + dv

    def fire_adj(g):
      if with_adj:
        pltpu.async_copy(wv.at[pl.ds(g * CHM, CHM)], adj_hbm.at[fidx.at[g]],
                         sadj)

    for b in range(NBUF):
      start_gather(bufs[b], gsems[b], b)

    @pl.loop(0, gpw, step=NBUF)
    def _(g):
      for b in range(NBUF):
        fire_adj(g + b)
        wait_gather(bufs[b], gsems[b])
        scale(bufs[b], g + b)
        pltpu.async_copy(bufs[b], acc_sh.at[didx.at[g + b]], ssems[b],
                         add=True)
      for b in range(NBUF):
        wait_scatter(bufs[b], ssems[b])
        start_gather(bufs[b], gsems[b], jnp.minimum(g + NBUF + b, gpw - 1))

    for b in range(NBUF):
      wait_gather(bufs[b], gsems[b])
    if with_adj:
      @pl.loop(0, gpw)
      def _(g):
        pltpu.make_async_copy(wv.at[pl.ds(0, CHM)], adj_hbm.at[fidx.at[0]],
                              sadj).wait()
    plsc.subcore_barrier()
    pltpu.sync_copy(acc_sh.at[pl.ds(s * rpt, rpt), :],
                    out_hbm.at[c, pl.ds(s * rpt, rpt), :])

  if with_adj:
    return k(hp, src2d, dst2d, ew, adj_ref)
  return k(hp, src2d, dst2d, ew)


def _tc_zeros(nn):
  blk = 524288
  grid = nn // blk

  def body(out_ref):
    out_ref[...] = jnp.zeros((blk,), jnp.float32)

  return pl.pallas_call(
      body,
      grid=(grid,),
      out_specs=pl.BlockSpec((blk,), lambda i: (i,)),
      out_shape=jax.ShapeDtypeStruct((nn,), jnp.float32),
  )()


def _tc_layer1(x, w1, b1, degp_t):
  n_nodes, d = x.shape
  r = 256
  grid = n_nodes // r

  def body(x_ref, w_ref, b_ref, deg_ref, hp_ref, dinv_ref):
    deg = jnp.sum(deg_ref[...], axis=1, keepdims=True) + 1.0
    dinv = jnp.where(deg > 0, lax.rsqrt(deg), 0.0)
    h = jnp.dot(x_ref[...], w_ref[...], preferred_element_type=jnp.float32)
    h = h + b_ref[...]
    hp_ref[...] = h * dinv
    dinv_ref[...] = dinv

  return pl.pallas_call(
      body,
      grid=(grid,),
      in_specs=[
          pl.BlockSpec((r, d), lambda i: (i, 0)),
          pl.BlockSpec((d, d), lambda i: (0, 0)),
          pl.BlockSpec((1, d), lambda i: (0, 0)),
          pl.BlockSpec((r, NC), lambda i: (i, 0)),
      ],
      out_specs=[
          pl.BlockSpec((r, d), lambda i: (i, 0)),
          pl.BlockSpec((r, 1), lambda i: (i, 0)),
      ],
      out_shape=[
          jax.ShapeDtypeStruct((n_nodes, d), jnp.float32),
          jax.ShapeDtypeStruct((n_nodes, 1), jnp.float32),
      ],
  )(x, w1, b1, degp_t)


def _tc_layer2(acc1, hp1, dinv, w2, b2):
  _, n_nodes, d = acc1.shape
  r = 256
  grid = n_nodes // r

  def body(acc_ref, hp_ref, dinv_ref, w_ref, b_ref, out_ref):
    dinv_blk = dinv_ref[...]
    h1 = dinv_blk * (acc_ref[0] + acc_ref[1] + hp_ref[...])
    x2 = jnp.maximum(h1, 0.0)
    h2 = jnp.dot(x2, w_ref[...], preferred_element_type=jnp.float32)
    h2 = h2 + b_ref[...]
    out_ref[...] = h2 * dinv_blk

  return pl.pallas_call(
      body,
      grid=(grid,),
      in_specs=[
          pl.BlockSpec((NC, r, d), lambda i: (0, i, 0)),
          pl.BlockSpec((r, d), lambda i: (i, 0)),
          pl.BlockSpec((r, 1), lambda i: (i, 0)),
          pl.BlockSpec((d, d), lambda i: (0, 0)),
          pl.BlockSpec((1, d), lambda i: (0, 0)),
      ],
      out_specs=pl.BlockSpec((r, d), lambda i: (i, 0)),
      out_shape=jax.ShapeDtypeStruct((n_nodes, d), jnp.float32),
  )(acc1, hp1, dinv, w2, b2)


def _tc_finalize(acc2, hp2, dinv, x):
  _, n_nodes, d = acc2.shape
  r = 256
  grid = n_nodes // r

  def body(acc_ref, hp_ref, dinv_ref, x_ref, z_ref, enc_ref):
    z = dinv_ref[...] * (acc_ref[0] + acc_ref[1] + hp_ref[...])
    z_ref[...] = z
    enc_ref[...] = z + x_ref[...]

  return pl.pallas_call(
      body,
      grid=(grid,),
      in_specs=[
          pl.BlockSpec((NC, r, d), lambda i: (0, i, 0)),
          pl.BlockSpec((r, d), lambda i: (i, 0)),
          pl.BlockSpec((r, 1), lambda i: (i, 0)),
          pl.BlockSpec((r, d), lambda i: (i, 0)),
      ],
      out_specs=[
          pl.BlockSpec((r, d), lambda i: (i, 0)),
          pl.BlockSpec((r, d), lambda i: (i, 0)),
      ],
      out_shape=[
          jax.ShapeDtypeStruct((n_nodes, d), jnp.float32),
          jax.ShapeDtypeStruct((n_nodes, d), jnp.float32),
      ],
  )(acc2, hp2, dinv, x)


def _tc_decoder(z, adj_flat):
  n_nodes, d = z.shape
  rb = 128
  grid = n_nodes // rb

  def body(z_ref, adj_ref, out_ref):
    i = pl.program_id(0)
    zr = z_ref[pl.ds(i * rb, rb), :]
    s = lax.dot_general(zr, z_ref[...],
                        dimension_numbers=(((1,), (1,)), ((), ())),
                        preferred_element_type=jnp.float32)
    adj = adj_ref[...].reshape(rb, n_nodes)
    out_ref[...] = jax.nn.sigmoid(s + adj)

  return pl.pallas_call(
      body,
      grid=(grid,),
      in_specs=[
          pl.BlockSpec((n_nodes, d), lambda i: (0, 0)),
          pl.BlockSpec((rb * n_nodes,), lambda i: (i,)),
      ],
      out_specs=pl.BlockSpec((rb, n_nodes), lambda i: (i, 0)),
      out_shape=jax.ShapeDtypeStruct((n_nodes, n_nodes), jnp.float32),
  )(z, adj_flat)


def kernel(node_features, edge_list, edge_attr, batch, W1, b1, W2, b2):
  n_nodes, d = node_features.shape
  src = edge_list[0]
  dst = edge_list[1]
  ew = edge_attr[:, 0]
  src2m = src.reshape(-1, CHM)
  dst2m = dst.reshape(-1, CHM)
  dst2d = dst.reshape(-1, CH)

  degp = _sc_degree(dst2d, ew, n_nodes)
  adj_ref = jax.new_ref(_tc_zeros(n_nodes * n_nodes))

  hp1, dinv = _tc_layer1(node_features, W1, b1.reshape(1, d),
                         degp.T)
  acc1 = _sc_msgpass(hp1, src2m, dst2m, ew, adj_ref)
  adj_flat = adj_ref[...]
  hp2 = _tc_layer2(acc1, hp1, dinv, W2, b2.reshape(1, d))
  acc2 = _sc_msgpass(hp2, src2m, dst2m, ew)
  z, encoded = _tc_finalize(acc2, hp2, dinv, node_features)
  probs = _tc_decoder(z, adj_flat)
  return (encoded, edge_list, probs)
